# Initial kernel scaffold; baseline (speedup 1.0000x reference)
import jax
import jax.numpy as jnp
from jax.experimental import pallas as pl


def kernel(x, idxs, pe):
    raise NotImplementedError("write your pallas kernel here")


# speedup vs baseline: 1.1372x; 1.1372x over previous
import functools

import jax
import jax.numpy as jnp
from jax import lax
from jax.experimental import pallas as pl
from jax.experimental.pallas import tpu as pltpu
from jax.experimental.pallas import tpu_sc as plsc

_NC = 2
_NS = 16
_NW = _NC * _NS


def _sc_gather_rows(pe, flat_idx):
    P = flat_idx.shape[0]
    D = pe.shape[1]
    rpw = P // _NW

    mesh = plsc.VectorSubcoreMesh(core_axis_name="c", subcore_axis_name="s")

    @functools.partial(
        pl.kernel,
        out_type=jax.ShapeDtypeStruct((P, D), jnp.float32),
        mesh=mesh,
        scratch_types=[
            pltpu.VMEM((rpw,), jnp.int32),
            pltpu.VMEM((rpw, D), jnp.float32),
            pltpu.SemaphoreType.DMA,
        ],
    )
    def gather_kernel(pe_hbm, idx_hbm, out_hbm, idx_v, rows_v, sem):
        wid = lax.axis_index("s") * _NC + lax.axis_index("c")
        base = wid * rpw
        pltpu.sync_copy(idx_hbm.at[pl.ds(base, rpw)], idx_v)
        pltpu.async_copy(pe_hbm.at[idx_v], rows_v, sem).wait()
        pltpu.sync_copy(rows_v, out_hbm.at[pl.ds(base, rpw)])

    return gather_kernel(pe, flat_idx)


def _combine_body(x_ref, u_ref, w_ref, o_ref):
    u = u_ref[0]
    w = w_ref[0]
    even_u = (lax.broadcasted_iota(jnp.int32, u.shape, 1) & 1) == 0
    even_w = (lax.broadcasted_iota(jnp.int32, w.shape, 1) & 1) == 0
    u_swap = jnp.where(even_u, jnp.roll(u, -1, axis=1), jnp.roll(u, 1, axis=1))
    wc = jnp.where(even_w, jnp.roll(w, -1, axis=1), w)
    ws = jnp.where(even_w, w, -jnp.roll(w, 1, axis=1))
    o_ref[0] = (
        x_ref[0]
        + u[:, None, :] * wc[None, :, :]
        + u_swap[:, None, :] * ws[None, :, :]
    )


def kernel(x, idxs, pe):
    B, N, D = x.shape
    L = idxs.shape[2]
    idxs = idxs.astype(jnp.int32)

    ab = (idxs[0][:, :, None] + idxs[1][:, None, :]).reshape(-1)
    cf = idxs[2].reshape(-1)
    n_real = ab.shape[0] + cf.shape[0]
    pad_to = -(-n_real // (8 * _NW)) * (8 * _NW)
    flat_idx = jnp.concatenate(
        [ab, cf, jnp.zeros((pad_to - n_real,), jnp.int32)]
    )

    rows = _sc_gather_rows(pe, flat_idx)
    U = rows[: B * L * L].reshape(B, L * L, D)
    W = rows[B * L * L : n_real].reshape(B, L, D)

    IJ_BLK = 32
    x4 = x.reshape(B, L * L, L, D)
    out = pl.pallas_call(
        _combine_body,
        grid=(B, (L * L) // IJ_BLK),
        in_specs=[
            pl.BlockSpec((1, IJ_BLK, L, D), lambda b, m: (b, m, 0, 0)),
            pl.BlockSpec((1, IJ_BLK, D), lambda b, m: (b, m, 0)),
            pl.BlockSpec((1, L, D), lambda b, m: (b, 0, 0)),
        ],
        out_specs=pl.BlockSpec((1, IJ_BLK, L, D), lambda b, m: (b, m, 0, 0)),
        out_shape=jax.ShapeDtypeStruct((B, L * L, L, D), jnp.float32),
    )(x4, U, W)
    return out.reshape(B, N, D)
